# baseline (speedup 1.0000x reference)
import functools

import jax
import jax.numpy as jnp
from jax import lax
from jax.experimental import pallas as pl
from jax.experimental.pallas import tpu as pltpu
from jax.experimental.pallas import tpu_sc as plsc

DIM = 64
OUT_D = 2 * DIM
GRP = 128
CHUNK_GRPS = 1
CHUNK = GRP * CHUNK_GRPS
NBUF = 2
LANE = 16
NC = 2
NS = 16
NW = NC * NS

BLK_C = 512
NROWS_T = 1000000
FULL_ROWS = (NROWS_T // GRP) * GRP
NBLK = FULL_ROWS // BLK_C
TAIL_ROWS = NROWS_T - FULL_ROWS
TAIL_PAIRS = TAIL_ROWS // 2


def _transpose_kernel():
  mesh = plsc.VectorSubcoreMesh(core_axis_name="c", subcore_axis_name="s")

  @functools.partial(
      pl.kernel,
      out_type=jax.ShapeDtypeStruct((NROWS_T * DIM,), jnp.float32),
      mesh=mesh,
      scratch_types=[
          pltpu.VMEM((DIM, BLK_C), jnp.float32),
          pltpu.VMEM((DIM, BLK_C), jnp.float32),
          pltpu.VMEM((BLK_C * DIM,), jnp.float32),
          pltpu.VMEM((TAIL_ROWS * DIM,), jnp.float32),
          pltpu.SemaphoreType.DMA,
          pltpu.SemaphoreType.DMA,
      ],
      compiler_params=pltpu.CompilerParams(needs_layout_passes=False),
  )
  def k(qtT, qtail, out, blk0, blk1, stg, tailv, sem0, sem1):
    wid = lax.axis_index("s") * NC + lax.axis_index("c")

    @pl.when(wid == 0)
    def _():
      pltpu.sync_copy(qtail, tailv)
      pltpu.sync_copy(tailv, out.at[pl.ds(FULL_ROWS * DIM, TAIL_ROWS * DIM)])

    blks = (blk0, blk1)
    sems = (sem0, sem1)
    iota64 = lax.iota(jnp.int32, LANE) * DIM

    def fire(b, buf, sem):
      pltpu.make_async_copy(
          qtT.at[:, pl.ds(b * BLK_C, BLK_C)], buf, sem).start()

    def drain(buf, sem):
      pltpu.make_async_copy(qtT.at[:, pl.ds(0, BLK_C)], buf, sem).wait()

    nblk_w = (NBLK - 1 - wid) // NW + 1
    for i in range(NBUF):
      @pl.when(i < nblk_w)
      def _():
        fire(wid + i * NW, blks[i], sems[i])

    def step(t, carry):
      bsel = lax.rem(t, NBUF)
      b = wid + t * NW

      def proc(blk, sem):
        drain(blk, sem)

        def col(ig, c2):
          i0 = ig * LANE
          base = jnp.broadcast_to(i0 * DIM, (LANE,)) + iota64
          for j in range(DIM):
            v = blk[j, pl.ds(i0, LANE)]
            plsc.store_scatter(stg, [base + j], v)
          return c2

        lax.fori_loop(0, BLK_C // LANE, col, 0)
        pltpu.sync_copy(
            stg, out.at[pl.ds(b * (BLK_C * DIM), BLK_C * DIM)])

        @pl.when(t + NBUF < nblk_w)
        def _():
          fire(b + NBUF * NW, blk, sem)

      @pl.when(bsel == 0)
      def _():
        proc(blk0, sem0)

      @pl.when(bsel == 1)
      def _():
        proc(blk1, sem1)

      return carry

    lax.fori_loop(0, nblk_w, step, 0)

  return k


def _gather_kernel(n_rows):
  rows_per_w = n_rows // NW
  grps_per_w = rows_per_w // GRP
  chunks_per_w = grps_per_w // CHUNK_GRPS
  assert grps_per_w % CHUNK_GRPS == 0 and chunks_per_w % NBUF == 0
  mesh = plsc.VectorSubcoreMesh(core_axis_name="c", subcore_axis_name="s")

  @functools.partial(
      pl.kernel,
      out_type=jax.ShapeDtypeStruct((n_rows, OUT_D), jnp.float32),
      mesh=mesh,
      scratch_types=[
          pltpu.VMEM((grps_per_w, GRP), jnp.int32),
          pltpu.VMEM((grps_per_w, GRP), jnp.int32),
          pltpu.VMEM((grps_per_w, GRP), jnp.int32),
          pltpu.VMEM((1, OUT_D), jnp.float32),
          pltpu.VMEM((CHUNK, OUT_D), jnp.float32),
          pltpu.VMEM((CHUNK, OUT_D), jnp.float32),
          pltpu.SemaphoreType.DMA,
          pltpu.SemaphoreType.DMA,
      ],
  )
  def k(qtab2, ctab2, qhalf, qoff, coff, out, qh_v, qo_v, co_v, ct_v,
        buf0, buf1, sem0, sem1):
    wid = lax.axis_index("s") * NC + lax.axis_index("c")
    rbase = wid * rows_per_w
    pltpu.sync_copy(qhalf.at[wid], qh_v)
    pltpu.sync_copy(qoff.at[wid], qo_v)
    pltpu.sync_copy(coff.at[wid], co_v)
    pltpu.sync_copy(ctab2, ct_v)
    bufs = (buf0, buf1)
    sems = (sem0, sem1)

    def fire(ch, buf, sem):
      g0 = ch * CHUNK_GRPS
      for j in range(CHUNK_GRPS):
        pltpu.make_async_copy(
            qtab2.at[qh_v.at[g0 + j]], buf.at[pl.ds(j * GRP, GRP)],
            sem).start()

    def drain(buf, sem):
      for j in range(CHUNK_GRPS):
        pltpu.make_async_copy(
            qtab2.at[qh_v.at[0]], buf.at[pl.ds(j * GRP, GRP)], sem).wait()

    def fixup(ch, buf):
      g0 = ch * CHUNK_GRPS
      for g in range(CHUNK_GRPS):
        def fix(gl, c2):
          qo16 = qo_v[g0 + g, pl.ds(gl * LANE, LANE)]
          co16 = co_v[g0 + g, pl.ds(gl * LANE, LANE)]
          r0 = g * GRP + gl * LANE
          for i in range(LANE):
            r = r0 + i
            p_off = qo16[i]
            c_off = co16[i]
            for kk in range(DIM // LANE):
              buf[r, pl.ds(kk * LANE, LANE)] = (
                  buf[r, pl.ds(p_off + kk * LANE, LANE)])
            for kk in range(DIM // LANE):
              buf[r, pl.ds(DIM + kk * LANE, LANE)] = (
                  ct_v[0, pl.ds(c_off + kk * LANE, LANE)])
          return c2

        lax.fori_loop(0, GRP // LANE, fix, 0)

    for b in range(NBUF):
      fire(b, bufs[b], sems[b])

    def step(ph, carry):
      for b in range(NBUF):
        ch = ph * NBUF + b
        drain(bufs[b], sems[b])
        fixup(ch, bufs[b])
        pltpu.sync_copy(bufs[b], out.at[pl.ds(rbase + ch * CHUNK, CHUNK)])

        @pl.when(ch + NBUF < chunks_per_w)
        def _():
          fire(ch + NBUF, bufs[b], sems[b])
      return carry

    lax.fori_loop(0, chunks_per_w // NBUF, step, 0)

  return k


@functools.partial(jax.jit, static_argnums=(6,))
def _embed(qtT, qtail, ctab2, qhalf, qoff, coff, n_rows):
  qtab1 = _transpose_kernel()(qtT, qtail)
  qtab2 = qtab1.reshape(NROWS_T // 2, OUT_D)
  return _gather_kernel(n_rows)(qtab2, ctab2, qhalf, qoff, coff)


def kernel(question_table, correctness_table, question_index, correctness_index):
  batch, hist = question_index.shape
  n_rows = batch * hist
  qtT = question_table.T
  qtail = question_table[FULL_ROWS:].reshape(TAIL_ROWS * DIM)
  ctab2 = correctness_table.reshape(1, OUT_D)
  qi = question_index.T.reshape(NW, -1, GRP).astype(jnp.int32)
  ci = correctness_index.T.reshape(NW, -1, GRP).astype(jnp.int32)
  qhalf = qi >> 1
  qoff = (qi & 1) * DIM
  coff = ci * DIM
  out = _embed(qtT, qtail, ctab2, qhalf, qoff, coff, n_rows)
  return out.reshape(hist, batch, OUT_D).transpose(1, 0, 2)


# speedup vs baseline: 1.9046x; 1.9046x over previous
import functools

import jax
import jax.numpy as jnp
from jax import lax
from jax.experimental import pallas as pl
from jax.experimental.pallas import tpu as pltpu
from jax.experimental.pallas import tpu_sc as plsc

DIM = 64
OUT_D = 2 * DIM
GRP = 128
CHUNK_GRPS = 1
CHUNK = GRP * CHUNK_GRPS
NBUF = 5
LANE = 16
NC = 2
NS = 16
NW = NC * NS


@functools.partial(jax.jit, static_argnums=(5,))
def _embed(qtab2, ctab2, qhalf, qoff, coff, n_rows):
  rows_per_w = n_rows // NW
  grps_per_w = rows_per_w // GRP
  chunks_per_w = grps_per_w // CHUNK_GRPS
  assert grps_per_w % CHUNK_GRPS == 0 and chunks_per_w % NBUF == 0
  mesh = plsc.VectorSubcoreMesh(core_axis_name="c", subcore_axis_name="s")

  @functools.partial(
      pl.kernel,
      out_type=jax.ShapeDtypeStruct((n_rows, OUT_D), jnp.float32),
      mesh=mesh,
      scratch_types=[
          pltpu.VMEM((grps_per_w, GRP), jnp.int32),
          pltpu.VMEM((grps_per_w, GRP), jnp.int32),
          pltpu.VMEM((grps_per_w, GRP), jnp.int32),
          pltpu.VMEM((1, OUT_D), jnp.float32),
      ] + [pltpu.VMEM((CHUNK, OUT_D), jnp.float32)] * NBUF
        + [pltpu.SemaphoreType.DMA] * (2 * NBUF),
  )
  def k(qtab2, ctab2, qhalf, qoff, coff, out, qh_v, qo_v, co_v, ct_v,
        *bufs_and_sems):
    bufs = bufs_and_sems[:NBUF]
    sems = bufs_and_sems[NBUF:2 * NBUF]
    wsems = bufs_and_sems[2 * NBUF:]
    wid = lax.axis_index("s") * NC + lax.axis_index("c")
    rbase = wid * rows_per_w
    pltpu.sync_copy(qhalf.at[wid], qh_v)
    pltpu.sync_copy(qoff.at[wid], qo_v)
    pltpu.sync_copy(coff.at[wid], co_v)
    pltpu.sync_copy(ctab2, ct_v)

    def fire(ch, buf, sem):
      g0 = ch * CHUNK_GRPS
      for j in range(CHUNK_GRPS):
        pltpu.make_async_copy(
            qtab2.at[qh_v.at[g0 + j]], buf.at[pl.ds(j * GRP, GRP)],
            sem).start()

    def drain(buf, sem):
      for j in range(CHUNK_GRPS):
        pltpu.make_async_copy(
            qtab2.at[qh_v.at[0]], buf.at[pl.ds(j * GRP, GRP)], sem).wait()

    def fixup(ch, buf):
      g0 = ch * CHUNK_GRPS
      for g in range(CHUNK_GRPS):
        def fix(gl, c2):
          qo16 = qo_v[g0 + g, pl.ds(gl * LANE, LANE)]
          co16 = co_v[g0 + g, pl.ds(gl * LANE, LANE)]
          r0 = g * GRP + gl * LANE
          for i in range(LANE):
            r = r0 + i
            p_off = qo16[i]
            c_off = co16[i]
            for kk in range(DIM // LANE):
              buf[r, pl.ds(kk * LANE, LANE)] = (
                  buf[r, pl.ds(p_off + kk * LANE, LANE)])
            for kk in range(DIM // LANE):
              buf[r, pl.ds(DIM + kk * LANE, LANE)] = (
                  ct_v[0, pl.ds(c_off + kk * LANE, LANE)])
          return c2

        lax.fori_loop(0, GRP // LANE, fix, 0)

    for b in range(NBUF):
      fire(b, bufs[b], sems[b])

    def wwait(ch, b):
      pltpu.make_async_copy(
          bufs[b], out.at[pl.ds(rbase + ch * CHUNK, CHUNK)], wsems[b]).wait()

    def step(ph, carry):
      for b in range(NBUF):
        ch = ph * NBUF + b
        drain(bufs[b], sems[b])
        fixup(ch, bufs[b])
        pltpu.make_async_copy(
            bufs[b], out.at[pl.ds(rbase + ch * CHUNK, CHUNK)],
            wsems[b]).start()
        pb = (b - 1) % NBUF
        pch = ch - 1

        def service():
          wwait(pch, pb)

          @pl.when(pch + NBUF < chunks_per_w)
          def _():
            fire(pch + NBUF, bufs[pb], sems[pb])

        if b == 0:
          @pl.when(ph > 0)
          def _():
            service()
        else:
          service()
      return carry

    lax.fori_loop(0, chunks_per_w // NBUF, step, 0)
    wwait(chunks_per_w - 1, NBUF - 1)

  return k(qtab2, ctab2, qhalf, qoff, coff)


def kernel(question_table, correctness_table, question_index, correctness_index):
  batch, hist = question_index.shape
  n_rows = batch * hist
  qtab2 = question_table.reshape(-1, OUT_D)
  ctab2 = correctness_table.reshape(1, OUT_D)
  qi = question_index.T.reshape(NW, -1, GRP).astype(jnp.int32)
  ci = correctness_index.T.reshape(NW, -1, GRP).astype(jnp.int32)
  qhalf = qi >> 1
  qoff = (qi & 1) * DIM
  coff = ci * DIM
  out = _embed(qtab2, ctab2, qhalf, qoff, coff, n_rows)
  return out.reshape(hist, batch, OUT_D).transpose(1, 0, 2)


# speedup vs baseline: 2.1432x; 1.1253x over previous
import functools

import jax
import jax.numpy as jnp
from jax import lax
from jax.experimental import pallas as pl
from jax.experimental.pallas import tpu as pltpu
from jax.experimental.pallas import tpu_sc as plsc

DIM = 64
OUT_D = 2 * DIM
GRP = 128
CHUNK_GRPS = 1
CHUNK = GRP * CHUNK_GRPS
NBUF = 5
LANE = 16
NC = 2
NS = 16
NW = NC * NS


@functools.partial(jax.jit, static_argnums=(5,))
def _embed(qtab2, ctab2, qhalf, qoff, coff, n_rows):
  rows_per_w = n_rows // NW
  grps_per_w = rows_per_w // GRP
  chunks_per_w = grps_per_w // CHUNK_GRPS
  assert grps_per_w % CHUNK_GRPS == 0 and chunks_per_w % NBUF == 0
  mesh = plsc.VectorSubcoreMesh(core_axis_name="c", subcore_axis_name="s")

  @functools.partial(
      pl.kernel,
      out_type=jax.ShapeDtypeStruct((n_rows, OUT_D), jnp.float32),
      mesh=mesh,
      scratch_types=[
          pltpu.VMEM((grps_per_w, GRP), jnp.int32),
          pltpu.VMEM((grps_per_w, GRP), jnp.int32),
          pltpu.VMEM((grps_per_w, GRP), jnp.int32),
          pltpu.VMEM((1, OUT_D), jnp.float32),
      ] + [pltpu.VMEM((CHUNK, OUT_D), jnp.float32)] * NBUF
        + [pltpu.SemaphoreType.DMA] * (2 * NBUF),
  )
  def k(qtab2, ctab2, qhalf, qoff, coff, out, qh_v, qo_v, co_v, ct_v,
        *bufs_and_sems):
    bufs = bufs_and_sems[:NBUF]
    sems = bufs_and_sems[NBUF:2 * NBUF]
    wsems = bufs_and_sems[2 * NBUF:]
    wid = lax.axis_index("s") * NC + lax.axis_index("c")
    rbase = wid * rows_per_w
    pltpu.sync_copy(qhalf.at[wid], qh_v)
    pltpu.sync_copy(qoff.at[wid], qo_v)
    pltpu.sync_copy(coff.at[wid], co_v)
    pltpu.sync_copy(ctab2, ct_v)

    def fire(ch, buf, sem):
      g0 = ch * CHUNK_GRPS
      for j in range(CHUNK_GRPS):
        pltpu.make_async_copy(
            qtab2.at[qh_v.at[g0 + j]], buf.at[pl.ds(j * GRP, GRP)],
            sem).start()

    def drain(buf, sem):
      for j in range(CHUNK_GRPS):
        pltpu.make_async_copy(
            qtab2.at[qh_v.at[0]], buf.at[pl.ds(j * GRP, GRP)], sem).wait()

    def fixup(ch, buf):
      g0 = ch * CHUNK_GRPS
      for g in range(CHUNK_GRPS):
        def fix(gl, c2):
          qo16 = qo_v[g0 + g, pl.ds(gl * LANE, LANE)]
          co16 = co_v[g0 + g, pl.ds(gl * LANE, LANE)]
          r0 = g * GRP + gl * LANE
          for i in range(LANE):
            r = r0 + i
            p_off = qo16[i]
            c_off = co16[i]

            @pl.when(p_off != 0)
            def _():
              for kk in range(DIM // LANE):
                buf[r, pl.ds(kk * LANE, LANE)] = (
                    buf[r, pl.ds(DIM + kk * LANE, LANE)])

            for kk in range(DIM // LANE):
              buf[r, pl.ds(DIM + kk * LANE, LANE)] = (
                  ct_v[0, pl.ds(c_off + kk * LANE, LANE)])
          return c2

        lax.fori_loop(0, GRP // LANE, fix, 0)

    for b in range(NBUF):
      fire(b, bufs[b], sems[b])

    def wwait(ch, b):
      pltpu.make_async_copy(
          bufs[b], out.at[pl.ds(rbase + ch * CHUNK, CHUNK)], wsems[b]).wait()

    def step(ph, carry):
      for b in range(NBUF):
        ch = ph * NBUF + b
        drain(bufs[b], sems[b])
        fixup(ch, bufs[b])
        pltpu.make_async_copy(
            bufs[b], out.at[pl.ds(rbase + ch * CHUNK, CHUNK)],
            wsems[b]).start()
        pb = (b - 1) % NBUF
        pch = ch - 1

        def service():
          wwait(pch, pb)

          @pl.when(pch + NBUF < chunks_per_w)
          def _():
            fire(pch + NBUF, bufs[pb], sems[pb])

        if b == 0:
          @pl.when(ph > 0)
          def _():
            service()
        else:
          service()
      return carry

    lax.fori_loop(0, chunks_per_w // NBUF, step, 0)
    wwait(chunks_per_w - 1, NBUF - 1)

  return k(qtab2, ctab2, qhalf, qoff, coff)


def kernel(question_table, correctness_table, question_index, correctness_index):
  batch, hist = question_index.shape
  n_rows = batch * hist
  qtab2 = question_table.reshape(-1, OUT_D)
  ctab2 = correctness_table.reshape(1, OUT_D)
  qi = question_index.T.reshape(NW, -1, GRP).astype(jnp.int32)
  ci = correctness_index.T.reshape(NW, -1, GRP).astype(jnp.int32)
  qhalf = qi >> 1
  qoff = (qi & 1) * DIM
  coff = ci * DIM
  out = _embed(qtab2, ctab2, qhalf, qoff, coff, n_rows)
  return out.reshape(hist, batch, OUT_D).transpose(1, 0, 2)
